# Initial kernel scaffold; baseline (speedup 1.0000x reference)
import jax
import jax.numpy as jnp
from jax.experimental import pallas as pl


def kernel(feat, knn_idx, W1, b1, W2, b2, gamma, beta):
    raise NotImplementedError("write your pallas kernel here")


# speedup vs baseline: 2.5240x; 2.5240x over previous
import functools

import jax
import jax.numpy as jnp
from jax import lax
from jax.experimental import pallas as pl
from jax.experimental.pallas import tpu as pltpu
from jax.experimental.pallas import tpu_sc as plsc

N, K, C = 10000, 32, 128
NK = N * K

_NC, _NS = 2, 16
_NW = _NC * _NS
_EPW = NK // _NW
_CH = 80
_NCH = _EPW // _CH

_BLK = 400
_GRID = N // _BLK

_INV_SQRT2 = 0.7071067811865476


def _gelu_exact(x):
    return 0.5 * x * (1.0 + lax.erf(x * _INV_SQRT2))


def _pre_body(feat_ref, w1_ref, b1_ref, a_ref, bv_ref):
    f = feat_ref[...]
    w1a = w1_ref[:C, :]
    w1b = w1_ref[C:, :]
    a_ref[...] = jnp.dot(f, w1b, preferred_element_type=jnp.float32,
                         precision=lax.Precision.HIGHEST)
    bv_ref[...] = jnp.dot(f, w1a - w1b, preferred_element_type=jnp.float32,
                          precision=lax.Precision.HIGHEST) + b1_ref[...]


def _pre_pass(feat, W1, b1):
    return pl.pallas_call(
        _pre_body,
        grid=(_GRID,),
        in_specs=[
            pl.BlockSpec((_BLK, C), lambda i: (i, 0)),
            pl.BlockSpec((2 * C, C), lambda i: (0, 0)),
            pl.BlockSpec((1, C), lambda i: (0, 0)),
        ],
        out_specs=[
            pl.BlockSpec((_BLK, C), lambda i: (i, 0)),
            pl.BlockSpec((_BLK, C), lambda i: (i, 0)),
        ],
        out_shape=[
            jax.ShapeDtypeStruct((N, C), jnp.float32),
            jax.ShapeDtypeStruct((N, C), jnp.float32),
        ],
    )(feat, W1, b1.reshape(1, C))


def _sc_gather_body(a_hbm, idx_hbm, g_hbm, idx_v, rows_v, sem):
    wid = lax.axis_index("s") * _NC + lax.axis_index("c")
    pltpu.sync_copy(idx_hbm.at[wid], idx_v)
    base = wid * _EPW

    def step(j, carry):
        pltpu.async_copy(a_hbm.at[idx_v.at[j]], rows_v, sem).wait()
        pltpu.sync_copy(rows_v, g_hbm.at[pl.ds(base + j * _CH, _CH)])
        return carry

    lax.fori_loop(0, _NCH, step, 0)


def _sc_gather(A, idx):
    mesh = plsc.VectorSubcoreMesh(core_axis_name="c", subcore_axis_name="s")
    return pl.kernel(
        _sc_gather_body,
        out_type=jax.ShapeDtypeStruct((NK, C), jnp.float32),
        mesh=mesh,
        scratch_types=[
            pltpu.VMEM((_NCH, _CH), jnp.int32),
            pltpu.VMEM((_CH, C), jnp.float32),
            pltpu.SemaphoreType.DMA,
        ],
    )(A, idx)


def _main_body(g_ref, bv_ref, feat_ref, w2_ref, b2_ref, gamma_ref, beta_ref,
               o_ref):
    bv = bv_ref[...]
    w2 = w2_ref[...]
    acc = None
    for k in range(K):
        h = _gelu_exact(bv + g_ref[:, k, :])
        hk = jnp.dot(h, w2, preferred_element_type=jnp.float32,
                     precision=lax.Precision.HIGHEST)
        acc = hk if acc is None else jnp.maximum(acc, hk)
    x = acc + b2_ref[...] + feat_ref[...]
    mean = jnp.mean(x, axis=1, keepdims=True)
    var = jnp.mean((x - mean) ** 2, axis=1, keepdims=True)
    o_ref[...] = ((x - mean) * lax.rsqrt(var + 1e-5)) * gamma_ref[...] \
        + beta_ref[...]


def _main_pass(G, Bv, feat, W2, b2, gamma, beta):
    return pl.pallas_call(
        _main_body,
        grid=(_GRID,),
        in_specs=[
            pl.BlockSpec((_BLK, K, C), lambda i: (i, 0, 0)),
            pl.BlockSpec((_BLK, C), lambda i: (i, 0)),
            pl.BlockSpec((_BLK, C), lambda i: (i, 0)),
            pl.BlockSpec((C, C), lambda i: (0, 0)),
            pl.BlockSpec((1, C), lambda i: (0, 0)),
            pl.BlockSpec((1, C), lambda i: (0, 0)),
            pl.BlockSpec((1, C), lambda i: (0, 0)),
        ],
        out_specs=pl.BlockSpec((_BLK, C), lambda i: (i, 0)),
        out_shape=jax.ShapeDtypeStruct((N, C), jnp.float32),
    )(G, Bv, feat, W2, b2.reshape(1, C), gamma.reshape(1, C),
      beta.reshape(1, C))


def kernel(feat, knn_idx, W1, b1, W2, b2, gamma, beta):
    idx = knn_idx.astype(jnp.int32).reshape(_NW, _NCH, _CH)
    A, Bv = _pre_pass(feat, W1, b1)
    G = _sc_gather(A, idx)
    return _main_pass(G.reshape(N, K, C), Bv, feat, W2, b2, gamma, beta)


# speedup vs baseline: 4.3039x; 1.7052x over previous
import functools

import jax
import jax.numpy as jnp
from jax import lax
from jax.experimental import pallas as pl
from jax.experimental.pallas import tpu as pltpu
from jax.experimental.pallas import tpu_sc as plsc

N, K, C = 10000, 32, 128
NK = N * K

_NC, _NS = 2, 16
_NW = _NC * _NS
_EPW = NK // _NW
_CH = 80
_NCH = _EPW // _CH
_NBUF = 5

_BLK = 400
_GRID = N // _BLK

_INV_SQRT2 = 0.7071067811865476


def _gelu_exact(x):
    return 0.5 * x * (1.0 + lax.erf(x * _INV_SQRT2))


def _pre_body(feat_ref, w1_ref, b1_ref, a_ref, bv_ref):
    f = feat_ref[...]
    w1a = w1_ref[:C, :]
    w1b = w1_ref[C:, :]
    a_ref[...] = jnp.dot(f, w1b, preferred_element_type=jnp.float32,
                         precision=lax.Precision.HIGHEST)
    bv_ref[...] = jnp.dot(f, w1a - w1b, preferred_element_type=jnp.float32,
                          precision=lax.Precision.HIGHEST) + b1_ref[...]


def _pre_pass(feat, W1, b1):
    return pl.pallas_call(
        _pre_body,
        grid=(_GRID,),
        in_specs=[
            pl.BlockSpec((_BLK, C), lambda i: (i, 0)),
            pl.BlockSpec((2 * C, C), lambda i: (0, 0)),
            pl.BlockSpec((1, C), lambda i: (0, 0)),
        ],
        out_specs=[
            pl.BlockSpec((_BLK, C), lambda i: (i, 0)),
            pl.BlockSpec((_BLK, C), lambda i: (i, 0)),
        ],
        out_shape=[
            jax.ShapeDtypeStruct((N, C), jnp.float32),
            jax.ShapeDtypeStruct((N, C), jnp.float32),
        ],
    )(feat, W1, b1.reshape(1, C))


def _sc_gather_body(a_hbm, idx_hbm, g_hbm, idx_v, r0, r1, r2, r3, r4,
                    si0, si1, si2, si3, si4, so0, so1, so2, so3, so4):
    rows = (r0, r1, r2, r3, r4)
    sin = (si0, si1, si2, si3, si4)
    sout = (so0, so1, so2, so3, so4)
    wid = lax.axis_index("s") * _NC + lax.axis_index("c")
    pltpu.sync_copy(idx_hbm.at[wid], idx_v)
    base = wid * _EPW

    pltpu.async_copy(a_hbm.at[idx_v.at[0]], rows[0], sin[0])
    pltpu.async_copy(a_hbm.at[idx_v.at[1]], rows[1], sin[1])

    def step(j, carry):
        c0 = j * _NBUF
        for b in range(_NBUF):
            c = c0 + b
            pltpu.make_async_copy(a_hbm.at[idx_v.at[c]], rows[b],
                                  sin[b]).wait()
            pltpu.async_copy(rows[b], g_hbm.at[pl.ds(base + c * _CH, _CH)],
                             sout[b])
            nb = (b + 2) % _NBUF
            nc = c + 2

            @pl.when(nc >= _NBUF)
            def _():
                pltpu.make_async_copy(
                    rows[nb], g_hbm.at[pl.ds(base + (c - 3) * _CH, _CH)],
                    sout[nb]).wait()

            @pl.when(nc < _NCH)
            def _():
                pltpu.async_copy(a_hbm.at[idx_v.at[nc]], rows[nb], sin[nb])
        return carry

    lax.fori_loop(0, _NCH // _NBUF, step, 0)
    for c in range(_NCH - (_NBUF - 2), _NCH):
        b = c % _NBUF
        pltpu.make_async_copy(rows[b], g_hbm.at[pl.ds(base + c * _CH, _CH)],
                              sout[b]).wait()


def _sc_gather(A, idx):
    mesh = plsc.VectorSubcoreMesh(core_axis_name="c", subcore_axis_name="s")
    return pl.kernel(
        _sc_gather_body,
        out_type=jax.ShapeDtypeStruct((NK, C), jnp.float32),
        mesh=mesh,
        scratch_types=(
            [pltpu.VMEM((_NCH, _CH), jnp.int32)]
            + [pltpu.VMEM((_CH, C), jnp.float32)] * _NBUF
            + [pltpu.SemaphoreType.DMA] * (2 * _NBUF)),
    )(A, idx)


def _main_body(g_ref, bv_ref, feat_ref, w2_ref, b2_ref, gamma_ref, beta_ref,
               o_ref):
    bv = bv_ref[...]
    w2 = w2_ref[...]
    acc = None
    for k in range(K):
        h = _gelu_exact(bv + g_ref[:, k, :])
        hk = jnp.dot(h, w2, preferred_element_type=jnp.float32,
                     precision=lax.Precision.DEFAULT)
        acc = hk if acc is None else jnp.maximum(acc, hk)
    x = acc + b2_ref[...] + feat_ref[...]
    mean = jnp.mean(x, axis=1, keepdims=True)
    var = jnp.mean((x - mean) ** 2, axis=1, keepdims=True)
    o_ref[...] = ((x - mean) * lax.rsqrt(var + 1e-5)) * gamma_ref[...] \
        + beta_ref[...]


def _main_pass(G, Bv, feat, W2, b2, gamma, beta):
    return pl.pallas_call(
        _main_body,
        grid=(_GRID,),
        in_specs=[
            pl.BlockSpec((_BLK, K, C), lambda i: (i, 0, 0)),
            pl.BlockSpec((_BLK, C), lambda i: (i, 0)),
            pl.BlockSpec((_BLK, C), lambda i: (i, 0)),
            pl.BlockSpec((C, C), lambda i: (0, 0)),
            pl.BlockSpec((1, C), lambda i: (0, 0)),
            pl.BlockSpec((1, C), lambda i: (0, 0)),
            pl.BlockSpec((1, C), lambda i: (0, 0)),
        ],
        out_specs=pl.BlockSpec((_BLK, C), lambda i: (i, 0)),
        out_shape=jax.ShapeDtypeStruct((N, C), jnp.float32),
    )(G, Bv, feat, W2, b2.reshape(1, C), gamma.reshape(1, C),
      beta.reshape(1, C))


def kernel(feat, knn_idx, W1, b1, W2, b2, gamma, beta):
    idx = knn_idx.astype(jnp.int32).reshape(_NW, _NCH, _CH)
    A, Bv = _pre_pass(feat, W1, b1)
    G = _sc_gather(A, idx)
    return _main_pass(G.reshape(N, K, C), Bv, feat, W2, b2, gamma, beta)


# speedup vs baseline: 4.9806x; 1.1572x over previous
import functools

import jax
import jax.numpy as jnp
from jax import lax
from jax.experimental import pallas as pl
from jax.experimental.pallas import tpu as pltpu
from jax.experimental.pallas import tpu_sc as plsc

N, K, C = 10000, 32, 128
NK = N * K

_NC, _NS = 2, 16
_NW = _NC * _NS
_CH = 80
_NBUF = 5

_BLK = 400

_SLABS = ((0, 3200), (3200, 3200), (6400, 3600))

_INV_SQRT2 = 0.7071067811865476


def _gelu_exact(x):
    return 0.5 * x * (1.0 + lax.erf(x * _INV_SQRT2))


def _pre_body(feat_ref, w1_ref, b1_ref, a_ref, bv_ref):
    f = feat_ref[...]
    w1a = w1_ref[:C, :]
    w1b = w1_ref[C:, :]
    a_ref[...] = jnp.dot(f, w1b, preferred_element_type=jnp.float32,
                         precision=lax.Precision.HIGHEST)
    bv_ref[...] = jnp.dot(f, w1a - w1b, preferred_element_type=jnp.float32,
                          precision=lax.Precision.HIGHEST) + b1_ref[...]


def _pre_pass(feat, W1, b1):
    return pl.pallas_call(
        _pre_body,
        grid=(N // _BLK,),
        in_specs=[
            pl.BlockSpec((_BLK, C), lambda i: (i, 0)),
            pl.BlockSpec((2 * C, C), lambda i: (0, 0)),
            pl.BlockSpec((1, C), lambda i: (0, 0)),
        ],
        out_specs=[
            pl.BlockSpec((_BLK, C), lambda i: (i, 0)),
            pl.BlockSpec((_BLK, C), lambda i: (i, 0)),
        ],
        out_shape=[
            jax.ShapeDtypeStruct((N, C), jnp.float32),
            jax.ShapeDtypeStruct((N, C), jnp.float32),
        ],
    )(feat, W1, b1.reshape(1, C))


def _sc_gather_body(nch, a_hbm, idx_hbm, g_hbm, idx_v, r0, r1, r2, r3, r4,
                    si0, si1, si2, si3, si4, so0, so1, so2, so3, so4):
    rows = (r0, r1, r2, r3, r4)
    sin = (si0, si1, si2, si3, si4)
    sout = (so0, so1, so2, so3, so4)
    epw = nch * _CH
    wid = lax.axis_index("s") * _NC + lax.axis_index("c")
    pltpu.sync_copy(idx_hbm.at[wid], idx_v)
    base = wid * epw

    pltpu.async_copy(a_hbm.at[idx_v.at[0]], rows[0], sin[0])
    pltpu.async_copy(a_hbm.at[idx_v.at[1]], rows[1], sin[1])

    def step(j, carry):
        c0 = j * _NBUF
        for b in range(_NBUF):
            c = c0 + b
            pltpu.make_async_copy(a_hbm.at[idx_v.at[c]], rows[b],
                                  sin[b]).wait()
            pltpu.async_copy(rows[b], g_hbm.at[pl.ds(base + c * _CH, _CH)],
                             sout[b])
            nb = (b + 2) % _NBUF
            nc = c + 2

            @pl.when(nc >= _NBUF)
            def _():
                pltpu.make_async_copy(
                    rows[nb], g_hbm.at[pl.ds(base + (c - 3) * _CH, _CH)],
                    sout[nb]).wait()

            @pl.when(nc < nch)
            def _():
                pltpu.async_copy(a_hbm.at[idx_v.at[nc]], rows[nb], sin[nb])
        return carry

    lax.fori_loop(0, nch // _NBUF, step, 0)
    for c in range(nch - (_NBUF - 2), nch):
        b = c % _NBUF
        pltpu.make_async_copy(rows[b], g_hbm.at[pl.ds(base + c * _CH, _CH)],
                              sout[b]).wait()


@functools.cache
def _sc_gather_call(nch):
    mesh = plsc.VectorSubcoreMesh(core_axis_name="c", subcore_axis_name="s")
    return pl.kernel(
        functools.partial(_sc_gather_body, nch),
        out_type=jax.ShapeDtypeStruct((_NW * nch * _CH, C), jnp.float32),
        mesh=mesh,
        scratch_types=(
            [pltpu.VMEM((nch, _CH), jnp.int32)]
            + [pltpu.VMEM((_CH, C), jnp.float32)] * _NBUF
            + [pltpu.SemaphoreType.DMA] * (2 * _NBUF)),
    )


def _sc_gather(A, idx_slab, n_nodes):
    nch = n_nodes // _CH
    idx = idx_slab.reshape(_NW, nch, _CH)
    return _sc_gather_call(nch)(A, idx)


def _main_body(g_ref, bv_ref, feat_ref, w2_ref, b2_ref, gamma_ref, beta_ref,
               o_ref):
    bv = bv_ref[...]
    w2 = w2_ref[...]
    acc = None
    for k in range(K):
        h = _gelu_exact(bv + g_ref[:, k, :])
        hk = jnp.dot(h, w2, preferred_element_type=jnp.float32,
                     precision=lax.Precision.DEFAULT)
        acc = hk if acc is None else jnp.maximum(acc, hk)
    x = acc + b2_ref[...] + feat_ref[...]
    mean = jnp.mean(x, axis=1, keepdims=True)
    var = jnp.mean((x - mean) ** 2, axis=1, keepdims=True)
    o_ref[...] = ((x - mean) * lax.rsqrt(var + 1e-5)) * gamma_ref[...] \
        + beta_ref[...]


def _main_pass(G, Bv, feat, W2, b2, gamma, beta, n0, nn):
    blk0 = n0 // _BLK
    return pl.pallas_call(
        _main_body,
        grid=(nn // _BLK,),
        in_specs=[
            pl.BlockSpec((_BLK, K, C), lambda i: (i, 0, 0)),
            pl.BlockSpec((_BLK, C), lambda i: (i + blk0, 0)),
            pl.BlockSpec((_BLK, C), lambda i: (i + blk0, 0)),
            pl.BlockSpec((C, C), lambda i: (0, 0)),
            pl.BlockSpec((1, C), lambda i: (0, 0)),
            pl.BlockSpec((1, C), lambda i: (0, 0)),
            pl.BlockSpec((1, C), lambda i: (0, 0)),
        ],
        out_specs=pl.BlockSpec((_BLK, C), lambda i: (i, 0)),
        out_shape=jax.ShapeDtypeStruct((nn, C), jnp.float32),
    )(G, Bv, feat, W2, b2.reshape(1, C), gamma.reshape(1, C),
      beta.reshape(1, C))


def kernel(feat, knn_idx, W1, b1, W2, b2, gamma, beta):
    idx = knn_idx.astype(jnp.int32).reshape(N, K)
    A, Bv = _pre_pass(feat, W1, b1)
    outs = []
    for n0, nn in _SLABS:
        G = _sc_gather(A, idx[n0:n0 + nn].reshape(-1), nn)
        outs.append(_main_pass(G.reshape(nn, K, C), Bv, feat, W2, b2,
                               gamma, beta, n0, nn))
    return jnp.concatenate(outs, axis=0)
